# Initial kernel scaffold; baseline (speedup 1.0000x reference)
import jax
import jax.numpy as jnp
from jax.experimental import pallas as pl


def kernel(x, edge_index, W1, b1, W2, b2, fc_w, fc_b):
    raise NotImplementedError("write your pallas kernel here")


# speedup vs baseline: 8.1787x; 8.1787x over previous
import functools

import jax
import jax.numpy as jnp
from jax import lax
from jax.experimental import pallas as pl
from jax.experimental.pallas import tpu as pltpu
from jax.experimental.pallas import tpu_sc as plsc

N = 10000
E = 320000
HID = 64
CHUNK = 128
ROWS = 2560
EPAD = ROWS * CHUNK - E
TRASH = N
NC, NS = 2, 16
NW = NC * NS
RPW = ROWS // NW
RPS = ROWS // NS
NPAD_R = 10400
HPAD = 128
NPAD_D = 10016
BR = 400
GRID = N // BR

@functools.lru_cache(maxsize=None)
def _mesh():
    return plsc.VectorSubcoreMesh(core_axis_name="c", subcore_axis_name="s")


def _deg_body(dst_hbm, z_hbm, deg_hbm, idxv, onesv, acc, sem):
    cid = lax.axis_index("c")
    sid = lax.axis_index("s")

    @pl.when(cid == 0)
    def _():
        pltpu.sync_copy(dst_hbm.at[pl.ds(sid * RPS, RPS)], idxv)
        for i in range(CHUNK // 16):
            onesv[pl.ds(i * 16, 16)] = jnp.full((16,), 1.0, jnp.float32)

        @pl.when(sid == 0)
        def _z():
            pltpu.sync_copy(z_hbm, acc)

        plsc.subcore_barrier()

        def step(j, c):
            pltpu.sync_copy(onesv, acc.at[idxv.at[j]], add=True)
            return c

        lax.fori_loop(0, RPS, step, 0)
        plsc.subcore_barrier()

        @pl.when(sid == 0)
        def _o():
            pltpu.sync_copy(acc, deg_hbm)


def _deg_call(dst2d, zdeg):
    return pl.kernel(
        _deg_body,
        out_type=jax.ShapeDtypeStruct((NPAD_D,), jnp.float32),
        mesh=_mesh(),
        scratch_types=[
            pltpu.VMEM((RPS, CHUNK), jnp.int32),
            pltpu.VMEM((CHUNK,), jnp.float32),
            pltpu.VMEM_SHARED((NPAD_D,), jnp.float32),
            pltpu.SemaphoreType.DMA,
        ],
    )(dst2d, zdeg)


def _scat_body(h_hbm, src_hbm, dst_hbm, z_hbm, out_hbm, srcv, dstv, rowbuf, acc, sem):
    cid = lax.axis_index("c")
    sid = lax.axis_index("s")
    wid = sid * NC + cid
    pltpu.sync_copy(src_hbm.at[pl.ds(wid * RPW, RPW)], srcv)
    pltpu.sync_copy(dst_hbm.at[pl.ds(wid * RPW, RPW)], dstv)

    @pl.when(sid == 0)
    def _z():
        pltpu.sync_copy(z_hbm, acc)

    plsc.subcore_barrier()

    def step(j, c):
        pltpu.async_copy(h_hbm.at[srcv.at[j]], rowbuf, sem).wait()
        pltpu.sync_copy(rowbuf, acc.at[dstv.at[j]], add=True)
        return c

    lax.fori_loop(0, RPW, step, 0)
    plsc.subcore_barrier()

    @pl.when(sid == 0)
    def _o():
        pltpu.sync_copy(acc, out_hbm.at[cid])


def _scat_call(h, src2d, dst2d, zrows):
    return pl.kernel(
        _scat_body,
        out_type=jax.ShapeDtypeStruct((NC, NPAD_R, HPAD), jnp.float32),
        mesh=_mesh(),
        scratch_types=[
            pltpu.VMEM((RPW, CHUNK), jnp.int32),
            pltpu.VMEM((RPW, CHUNK), jnp.int32),
            pltpu.VMEM((CHUNK, HPAD), jnp.float32),
            pltpu.VMEM_SHARED((NPAD_R, HPAD), jnp.float32),
            pltpu.SemaphoreType.DMA,
        ],
    )(h, src2d, dst2d, zrows)


def _mm_body(x_ref, w_ref, o_ref):
    o_ref[...] = jnp.dot(x_ref[...], w_ref[...],
                         preferred_element_type=jnp.float32)


def _mm_call(x, w):
    k = x.shape[1]
    return pl.pallas_call(
        _mm_body,
        grid=(GRID,),
        in_specs=[
            pl.BlockSpec((BR, k), lambda i: (i, 0)),
            pl.BlockSpec((k, HID), lambda i: (0, 0)),
        ],
        out_specs=pl.BlockSpec((BR, HID), lambda i: (i, 0)),
        out_shape=jax.ShapeDtypeStruct((N, HID), jnp.float32),
    )(x, w)


def _d_body(p_ref, deg_ref, hp_ref, dis_ref):
    dis = lax.rsqrt(deg_ref[...] + 1.0)
    dis_ref[...] = dis
    hp_ref[:, :HID] = p_ref[...] * dis
    hp_ref[:, HID:] = jnp.zeros((BR, HPAD - HID), jnp.float32)


def _d_call(p1, degc):
    return pl.pallas_call(
        _d_body,
        grid=(GRID,),
        in_specs=[
            pl.BlockSpec((BR, HID), lambda i: (i, 0)),
            pl.BlockSpec((BR, 1), lambda i: (i, 0)),
        ],
        out_specs=[
            pl.BlockSpec((BR, HPAD), lambda i: (i, 0)),
            pl.BlockSpec((BR, 1), lambda i: (i, 0)),
        ],
        out_shape=[
            jax.ShapeDtypeStruct((N, HPAD), jnp.float32),
            jax.ShapeDtypeStruct((N, 1), jnp.float32),
        ],
    )(p1, degc)


def _c1_body(parts_ref, hp_ref, dis_ref, b_ref, w_ref, o_ref):
    s = (parts_ref[0, :, :HID] + parts_ref[1, :, :HID] + hp_ref[:, :HID])
    dis = dis_ref[...]
    h = jnp.maximum(dis * s + b_ref[...], 0.0)
    o_ref[:, :HID] = dis * jnp.dot(h, w_ref[...],
                                   preferred_element_type=jnp.float32)
    o_ref[:, HID:] = jnp.zeros((BR, HPAD - HID), jnp.float32)


def _c1_call(parts, hp, dis, b, w):
    return pl.pallas_call(
        _c1_body,
        grid=(GRID,),
        in_specs=[
            pl.BlockSpec((NC, BR, HPAD), lambda i: (0, i, 0)),
            pl.BlockSpec((BR, HPAD), lambda i: (i, 0)),
            pl.BlockSpec((BR, 1), lambda i: (i, 0)),
            pl.BlockSpec((1, HID), lambda i: (0, 0)),
            pl.BlockSpec((HID, HID), lambda i: (0, 0)),
        ],
        out_specs=pl.BlockSpec((BR, HPAD), lambda i: (i, 0)),
        out_shape=jax.ShapeDtypeStruct((N, HPAD), jnp.float32),
    )(parts, hp, dis, b, w)


def _c2_body(parts_ref, hp_ref, dis_ref, b_ref, fw_ref, fb_ref, o_ref):
    s = (parts_ref[0, :, :HID] + parts_ref[1, :, :HID] + hp_ref[:, :HID])
    dis = dis_ref[...]
    h = jnp.maximum(dis * s + b_ref[...], 0.0)
    o_ref[...] = jnp.sum(h * fw_ref[...], axis=1, keepdims=True) + fb_ref[...]


def _c2_call(parts, hp, dis, b, fw, fb):
    return pl.pallas_call(
        _c2_body,
        grid=(GRID,),
        in_specs=[
            pl.BlockSpec((NC, BR, HPAD), lambda i: (0, i, 0)),
            pl.BlockSpec((BR, HPAD), lambda i: (i, 0)),
            pl.BlockSpec((BR, 1), lambda i: (i, 0)),
            pl.BlockSpec((1, HID), lambda i: (0, 0)),
            pl.BlockSpec((1, HID), lambda i: (0, 0)),
            pl.BlockSpec((1, 1), lambda i: (0, 0)),
        ],
        out_specs=pl.BlockSpec((BR, 1), lambda i: (i, 0)),
        out_shape=jax.ShapeDtypeStruct((N, 1), jnp.float32),
    )(parts, hp, dis, b, fw, fb)


def kernel(x, edge_index, W1, b1, W2, b2, fc_w, fc_b):
    ei = edge_index.astype(jnp.int32)
    src2d = jnp.concatenate(
        [ei[0], jnp.zeros((EPAD,), jnp.int32)]).reshape(ROWS, CHUNK)
    dst2d = jnp.concatenate(
        [ei[1], jnp.full((EPAD,), TRASH, jnp.int32)]).reshape(ROWS, CHUNK)
    zdeg = jnp.zeros((NPAD_D,), jnp.float32)
    zrows = jnp.zeros((NPAD_R, HPAD), jnp.float32)

    deg = _deg_call(dst2d, zdeg)
    p1 = _mm_call(x, W1)
    degc = deg[:N].reshape(N, 1)
    h1p, dis = _d_call(p1, degc)
    parts1 = _scat_call(h1p, src2d, dst2d, zrows)
    h2p = _c1_call(parts1, h1p, dis,
                   b1.reshape(1, HID), W2)
    parts2 = _scat_call(h2p, src2d, dst2d, zrows)
    out = _c2_call(parts2, h2p, dis, b2.reshape(1, HID),
                   fc_w.reshape(1, HID), fc_b.reshape(1, 1))
    return out


# speedup vs baseline: 8.7754x; 1.0730x over previous
import functools

import jax
import jax.numpy as jnp
from jax import lax
from jax.experimental import pallas as pl
from jax.experimental.pallas import tpu as pltpu
from jax.experimental.pallas import tpu_sc as plsc

N = 10000
E = 320000
HID = 64
CHUNK = 128
ROWS = 2560
IB = 16
EPAD = ROWS * CHUNK - E
TRASH = N
NC, NS = 2, 16
NW = NC * NS
RPW = ROWS // NW
RPS = ROWS // NS
NPAD_R = 10400
HPAD = 128
NPAD_D = 10016
BR = 400
GRID = N // BR

@functools.lru_cache(maxsize=None)
def _mesh():
    return plsc.VectorSubcoreMesh(core_axis_name="c", subcore_axis_name="s")


def _deg_body(dst_hbm, z_hbm, deg_hbm, idxv, onesv, acc, sem):
    cid = lax.axis_index("c")
    sid = lax.axis_index("s")

    @pl.when(cid == 0)
    def _():
        pltpu.sync_copy(dst_hbm.at[pl.ds(sid * RPS, RPS)], idxv)
        for i in range(CHUNK // 16):
            onesv[pl.ds(i * 16, 16)] = jnp.full((16,), 1.0, jnp.float32)

        @pl.when(sid == 0)
        def _z():
            pltpu.sync_copy(z_hbm, acc)

        plsc.subcore_barrier()

        def step(j, c):
            pltpu.sync_copy(onesv, acc.at[idxv.at[j]], add=True)
            return c

        lax.fori_loop(0, RPS, step, 0)
        plsc.subcore_barrier()

        @pl.when(sid == 0)
        def _o():
            pltpu.sync_copy(acc, deg_hbm)


def _deg_call(dst2d, zdeg):
    return pl.kernel(
        _deg_body,
        out_type=jax.ShapeDtypeStruct((NPAD_D,), jnp.float32),
        mesh=_mesh(),
        scratch_types=[
            pltpu.VMEM((RPS, CHUNK), jnp.int32),
            pltpu.VMEM((CHUNK,), jnp.float32),
            pltpu.VMEM_SHARED((NPAD_D,), jnp.float32),
            pltpu.SemaphoreType.DMA,
        ],
    )(dst2d, zdeg)


def _scat_body(h_hbm, src_hbm, dst_hbm, z_hbm, out_hbm,
               srcv, dstv, buf0, buf1, acc, sem0, sem1):
    cid = lax.axis_index("c")
    sid = lax.axis_index("s")
    wid = sid * NC + cid

    @pl.when(sid == 0)
    def _z():
        pltpu.sync_copy(z_hbm, acc)

    plsc.subcore_barrier()

    def stage(s, cs):
        r0 = wid * RPW + s * IB
        pltpu.sync_copy(src_hbm.at[pl.ds(r0, IB)], srcv)
        pltpu.sync_copy(dst_hbm.at[pl.ds(r0, IB)], dstv)
        pltpu.async_copy(h_hbm.at[srcv.at[0]], buf0, sem0)

        def step(i, c):
            j0 = 2 * i
            pltpu.make_async_copy(h_hbm.at[srcv.at[j0]], buf0, sem0).wait()
            pltpu.async_copy(h_hbm.at[srcv.at[j0 + 1]], buf1, sem1)
            pltpu.sync_copy(buf0, acc.at[dstv.at[j0]], add=True)
            pltpu.make_async_copy(h_hbm.at[srcv.at[j0 + 1]], buf1, sem1).wait()

            @pl.when(i < IB // 2 - 1)
            def _pf():
                pltpu.async_copy(h_hbm.at[srcv.at[j0 + 2]], buf0, sem0)

            pltpu.sync_copy(buf1, acc.at[dstv.at[j0 + 1]], add=True)
            return c

        lax.fori_loop(0, IB // 2, step, 0)
        return cs

    lax.fori_loop(0, RPW // IB, stage, 0)
    plsc.subcore_barrier()

    @pl.when(sid == 0)
    def _o():
        pltpu.sync_copy(acc, out_hbm.at[cid])


def _scat_call(h, src2d, dst2d, zrows):
    return pl.kernel(
        _scat_body,
        out_type=jax.ShapeDtypeStruct((NC, NPAD_R, HPAD), jnp.float32),
        mesh=_mesh(),
        scratch_types=[
            pltpu.VMEM((IB, CHUNK), jnp.int32),
            pltpu.VMEM((IB, CHUNK), jnp.int32),
            pltpu.VMEM((CHUNK, HPAD), jnp.float32),
            pltpu.VMEM((CHUNK, HPAD), jnp.float32),
            pltpu.VMEM_SHARED((NPAD_R, HPAD), jnp.float32),
            pltpu.SemaphoreType.DMA,
            pltpu.SemaphoreType.DMA,
        ],
    )(h, src2d, dst2d, zrows)


def _mm_body(x_ref, w_ref, o_ref):
    o_ref[...] = jnp.dot(x_ref[...], w_ref[...],
                         preferred_element_type=jnp.float32)


def _mm_call(x, w):
    k = x.shape[1]
    return pl.pallas_call(
        _mm_body,
        grid=(GRID,),
        in_specs=[
            pl.BlockSpec((BR, k), lambda i: (i, 0)),
            pl.BlockSpec((k, HID), lambda i: (0, 0)),
        ],
        out_specs=pl.BlockSpec((BR, HID), lambda i: (i, 0)),
        out_shape=jax.ShapeDtypeStruct((N, HID), jnp.float32),
    )(x, w)


def _d_body(p_ref, deg_ref, hp_ref, dis_ref):
    dis = lax.rsqrt(deg_ref[...] + 1.0)
    dis_ref[...] = dis
    hp_ref[:, :HID] = p_ref[...] * dis
    hp_ref[:, HID:] = jnp.zeros((BR, HPAD - HID), jnp.float32)


def _d_call(p1, degc):
    return pl.pallas_call(
        _d_body,
        grid=(GRID,),
        in_specs=[
            pl.BlockSpec((BR, HID), lambda i: (i, 0)),
            pl.BlockSpec((BR, 1), lambda i: (i, 0)),
        ],
        out_specs=[
            pl.BlockSpec((BR, HPAD), lambda i: (i, 0)),
            pl.BlockSpec((BR, 1), lambda i: (i, 0)),
        ],
        out_shape=[
            jax.ShapeDtypeStruct((N, HPAD), jnp.float32),
            jax.ShapeDtypeStruct((N, 1), jnp.float32),
        ],
    )(p1, degc)


def _c1_body(parts_ref, hp_ref, dis_ref, b_ref, w_ref, o_ref):
    s = (parts_ref[0, :, :HID] + parts_ref[1, :, :HID] + hp_ref[:, :HID])
    dis = dis_ref[...]
    h = jnp.maximum(dis * s + b_ref[...], 0.0)
    o_ref[:, :HID] = dis * jnp.dot(h, w_ref[...],
                                   preferred_element_type=jnp.float32)
    o_ref[:, HID:] = jnp.zeros((BR, HPAD - HID), jnp.float32)


def _c1_call(parts, hp, dis, b, w):
    return pl.pallas_call(
        _c1_body,
        grid=(GRID,),
        in_specs=[
            pl.BlockSpec((NC, BR, HPAD), lambda i: (0, i, 0)),
            pl.BlockSpec((BR, HPAD), lambda i: (i, 0)),
            pl.BlockSpec((BR, 1), lambda i: (i, 0)),
            pl.BlockSpec((1, HID), lambda i: (0, 0)),
            pl.BlockSpec((HID, HID), lambda i: (0, 0)),
        ],
        out_specs=pl.BlockSpec((BR, HPAD), lambda i: (i, 0)),
        out_shape=jax.ShapeDtypeStruct((N, HPAD), jnp.float32),
    )(parts, hp, dis, b, w)


def _c2_body(parts_ref, hp_ref, dis_ref, b_ref, fw_ref, fb_ref, o_ref):
    s = (parts_ref[0, :, :HID] + parts_ref[1, :, :HID] + hp_ref[:, :HID])
    dis = dis_ref[...]
    h = jnp.maximum(dis * s + b_ref[...], 0.0)
    o_ref[...] = jnp.sum(h * fw_ref[...], axis=1, keepdims=True) + fb_ref[...]


def _c2_call(parts, hp, dis, b, fw, fb):
    return pl.pallas_call(
        _c2_body,
        grid=(GRID,),
        in_specs=[
            pl.BlockSpec((NC, BR, HPAD), lambda i: (0, i, 0)),
            pl.BlockSpec((BR, HPAD), lambda i: (i, 0)),
            pl.BlockSpec((BR, 1), lambda i: (i, 0)),
            pl.BlockSpec((1, HID), lambda i: (0, 0)),
            pl.BlockSpec((1, HID), lambda i: (0, 0)),
            pl.BlockSpec((1, 1), lambda i: (0, 0)),
        ],
        out_specs=pl.BlockSpec((BR, 1), lambda i: (i, 0)),
        out_shape=jax.ShapeDtypeStruct((N, 1), jnp.float32),
    )(parts, hp, dis, b, fw, fb)


def kernel(x, edge_index, W1, b1, W2, b2, fc_w, fc_b):
    ei = edge_index.astype(jnp.int32)
    src2d = jnp.concatenate(
        [ei[0], jnp.zeros((EPAD,), jnp.int32)]).reshape(ROWS, CHUNK)
    dst2d = jnp.concatenate(
        [ei[1], jnp.full((EPAD,), TRASH, jnp.int32)]).reshape(ROWS, CHUNK)
    zdeg = jnp.zeros((NPAD_D,), jnp.float32)
    zrows = jnp.zeros((NPAD_R, HPAD), jnp.float32)

    deg = _deg_call(dst2d, zdeg)
    p1 = _mm_call(x, W1)
    degc = deg[:N].reshape(N, 1)
    h1p, dis = _d_call(p1, degc)
    parts1 = _scat_call(h1p, src2d, dst2d, zrows)
    h2p = _c1_call(parts1, h1p, dis,
                   b1.reshape(1, HID), W2)
    parts2 = _scat_call(h2p, src2d, dst2d, zrows)
    out = _c2_call(parts2, h2p, dis, b2.reshape(1, HID),
                   fc_w.reshape(1, HID), fc_b.reshape(1, 1))
    return out
